# baseline (device time: 16441 ns/iter reference)
import jax
import jax.numpy as jnp
from jax import lax
from jax.experimental import pallas as pl
from jax.experimental.pallas import tpu as pltpu

N_DEV = 16


def kernel(x, w_mat):
    k_total, k_per = x.shape
    m_per = k_total // N_DEV
    n = w_mat.shape[1]

    def body(x_hbm_ref, w_hbm_ref, out_hbm_ref, x_ref, w_ref, out_ref,
             comm_ref, send_sems, recv_sems, ready_sems, w_sem, x_sem,
             out_sem):
        my = lax.axis_index("i")

        x_load = pltpu.make_async_copy(x_hbm_ref, x_ref, x_sem)
        x_load.start()
        k_chunk = k_total // 4
        w_loads = []
        for g in range(4):
            w_load = pltpu.make_async_copy(
                w_hbm_ref.at[pl.ds(g * k_chunk, k_chunk), :],
                w_ref.at[pl.ds(g * k_chunk, k_chunk), :],
                w_sem.at[g],
            )
            w_load.start()
            w_loads.append(w_load)

        barrier_sem = pltpu.get_barrier_semaphore()
        pl.semaphore_signal(barrier_sem, inc=1)
        pl.semaphore_wait(barrier_sem, 1)

        for d in range(1, N_DEV):
            pl.semaphore_signal(
                ready_sems.at[15 - d], inc=1,
                device_id=(lax.rem(my + d, N_DEV),),
                device_id_type=pl.DeviceIdType.MESH,
            )

        x_load.wait()
        comm_ref[my] = x_ref[pl.ds(my * m_per, m_per), :]

        rdmas = []
        for d in range(1, N_DEV):
            tgt = lax.rem(my + d, N_DEV)
            rdma = pltpu.make_async_remote_copy(
                src_ref=x_ref.at[pl.ds(tgt * m_per, m_per), :],
                dst_ref=comm_ref.at[my],
                send_sem=send_sems.at[d - 1],
                recv_sem=recv_sems.at[my],
                device_id=(tgt,),
                device_id_type=pl.DeviceIdType.MESH,
            )
            pl.semaphore_wait(ready_sems.at[d - 1], 1)
            rdma.start()
            rdmas.append(rdma)

        for g in range(4):
            for s in range(4 * g, 4 * g + 4):
                @pl.when(s != my)
                def _():
                    recv = pltpu.make_async_remote_copy(
                        src_ref=x_ref.at[pl.ds(0, m_per), :],
                        dst_ref=comm_ref.at[s],
                        send_sem=send_sems.at[0],
                        recv_sem=recv_sems.at[s],
                        device_id=(my,),
                        device_id_type=pl.DeviceIdType.MESH,
                    )
                    recv.wait_recv()
            w_loads[g].wait()
            xg = jnp.concatenate(
                [comm_ref[s] for s in range(4 * g, 4 * g + 4)], axis=1
            )
            partial = jnp.dot(
                xg,
                w_ref[pl.ds(g * k_chunk, k_chunk), :],
                preferred_element_type=jnp.float32,
            )
            if g == 0:
                out_ref[:, :] = partial
            else:
                out_ref[:, :] += partial

        out_store = pltpu.make_async_copy(out_ref, out_hbm_ref, out_sem)
        out_store.start()

        for d in range(1, N_DEV):
            rdmas[d - 1].wait_send()
        out_store.wait()

    return pl.pallas_call(
        body,
        out_shape=jax.ShapeDtypeStruct((m_per, n), jnp.float32),
        in_specs=[
            pl.BlockSpec(memory_space=pl.ANY),
            pl.BlockSpec(memory_space=pl.ANY),
        ],
        out_specs=pl.BlockSpec(memory_space=pl.ANY),
        scratch_shapes=[
            pltpu.VMEM((k_total, k_per), x.dtype),
            pltpu.VMEM((k_total, n), w_mat.dtype),
            pltpu.VMEM((m_per, n), jnp.float32),
            pltpu.VMEM((N_DEV, m_per, k_per), x.dtype),
            pltpu.SemaphoreType.DMA((N_DEV - 1,)),
            pltpu.SemaphoreType.DMA((N_DEV,)),
            pltpu.SemaphoreType.REGULAR((N_DEV - 1,)),
            pltpu.SemaphoreType.DMA((4,)),
            pltpu.SemaphoreType.DMA,
            pltpu.SemaphoreType.DMA,
        ],
        compiler_params=pltpu.CompilerParams(collective_id=0),
    )(x, w_mat)


# device time: 16308 ns/iter; 1.0082x vs baseline; 1.0082x over previous
import jax
import jax.numpy as jnp
from jax import lax
from jax.experimental import pallas as pl
from jax.experimental.pallas import tpu as pltpu

N_DEV = 16


def kernel(x, w_mat):
    k_total, k_per = x.shape
    m_per = k_total // N_DEV
    n = w_mat.shape[1]

    def body(x_hbm_ref, w_hbm_ref, out_hbm_ref, x_ref, w_ref, out_ref,
             comm_ref, send_sems, recv_sems, w_sem, x_sem, out_sem):
        my = lax.axis_index("i")

        x_load = pltpu.make_async_copy(x_hbm_ref, x_ref, x_sem)
        x_load.start()
        k_chunk = k_total // 4
        w_loads = []
        for g in range(4):
            w_load = pltpu.make_async_copy(
                w_hbm_ref.at[pl.ds(g * k_chunk, k_chunk), :],
                w_ref.at[pl.ds(g * k_chunk, k_chunk), :],
                w_sem.at[g],
            )
            w_load.start()
            w_loads.append(w_load)

        barrier_sem = pltpu.get_barrier_semaphore()
        for d in range(1, N_DEV):
            pl.semaphore_signal(
                barrier_sem, inc=1,
                device_id=(lax.rem(my + d, N_DEV),),
                device_id_type=pl.DeviceIdType.MESH,
            )

        x_load.wait()
        comm_ref[my] = x_ref[pl.ds(my * m_per, m_per), :]

        pl.semaphore_wait(barrier_sem, N_DEV - 1)

        rdmas = []
        for d in range(1, N_DEV):
            tgt = lax.rem(my + d, N_DEV)
            rdma = pltpu.make_async_remote_copy(
                src_ref=x_ref.at[pl.ds(tgt * m_per, m_per), :],
                dst_ref=comm_ref.at[my],
                send_sem=send_sems.at[d - 1],
                recv_sem=recv_sems.at[my],
                device_id=(tgt,),
                device_id_type=pl.DeviceIdType.MESH,
            )
            rdma.start()
            rdmas.append(rdma)

        for g in range(4):
            for s in range(4 * g, 4 * g + 4):
                @pl.when(s != my)
                def _():
                    recv = pltpu.make_async_remote_copy(
                        src_ref=x_ref.at[pl.ds(0, m_per), :],
                        dst_ref=comm_ref.at[s],
                        send_sem=send_sems.at[0],
                        recv_sem=recv_sems.at[s],
                        device_id=(my,),
                        device_id_type=pl.DeviceIdType.MESH,
                    )
                    recv.wait_recv()
            w_loads[g].wait()
            xg = jnp.concatenate(
                [comm_ref[s] for s in range(4 * g, 4 * g + 4)], axis=1
            )
            partial = jnp.dot(
                xg,
                w_ref[pl.ds(g * k_chunk, k_chunk), :],
                preferred_element_type=jnp.float32,
            )
            if g == 0:
                out_ref[:, :] = partial
            else:
                out_ref[:, :] += partial

        out_store = pltpu.make_async_copy(out_ref, out_hbm_ref, out_sem)
        out_store.start()

        for d in range(1, N_DEV):
            rdmas[d - 1].wait_send()
        out_store.wait()

    return pl.pallas_call(
        body,
        out_shape=jax.ShapeDtypeStruct((m_per, n), jnp.float32),
        in_specs=[
            pl.BlockSpec(memory_space=pl.ANY),
            pl.BlockSpec(memory_space=pl.ANY),
        ],
        out_specs=pl.BlockSpec(memory_space=pl.ANY),
        scratch_shapes=[
            pltpu.VMEM((k_total, k_per), x.dtype),
            pltpu.VMEM((k_total, n), w_mat.dtype),
            pltpu.VMEM((m_per, n), jnp.float32),
            pltpu.VMEM((N_DEV, m_per, k_per), x.dtype),
            pltpu.SemaphoreType.DMA((N_DEV - 1,)),
            pltpu.SemaphoreType.DMA((N_DEV,)),
            pltpu.SemaphoreType.DMA((4,)),
            pltpu.SemaphoreType.DMA,
            pltpu.SemaphoreType.DMA,
        ],
        compiler_params=pltpu.CompilerParams(collective_id=0),
    )(x, w_mat)


# device time: 15925 ns/iter; 1.0324x vs baseline; 1.0241x over previous
import jax
import jax.numpy as jnp
from jax import lax
from jax.experimental import pallas as pl
from jax.experimental.pallas import tpu as pltpu

N_DEV = 16


def kernel(x, w_mat):
    k_total, k_per = x.shape
    m_per = k_total // N_DEV
    n = w_mat.shape[1]

    def body(x_hbm_ref, w_hbm_ref, out_hbm_ref, x_ref, w_ref, out_ref,
             comm_ref, send_sems, recv_sems, w_sem, x_sem, out_sem):
        my = lax.axis_index("i")

        barrier_sem = pltpu.get_barrier_semaphore()
        for d in range(1, N_DEV):
            pl.semaphore_signal(
                barrier_sem, inc=1,
                device_id=(lax.rem(my + d, N_DEV),),
                device_id_type=pl.DeviceIdType.MESH,
            )

        x_load = pltpu.make_async_copy(x_hbm_ref, x_ref, x_sem)
        x_load.start()
        k_chunk = k_total // 4
        w_loads = []
        for g in range(4):
            w_load = pltpu.make_async_copy(
                w_hbm_ref.at[pl.ds(g * k_chunk, k_chunk), :],
                w_ref.at[pl.ds(g * k_chunk, k_chunk), :],
                w_sem.at[g],
            )
            w_load.start()
            w_loads.append(w_load)

        x_load.wait()
        comm_ref[my] = x_ref[pl.ds(my * m_per, m_per), :]

        pl.semaphore_wait(barrier_sem, N_DEV - 1)

        rdmas = []
        for d in range(1, N_DEV):
            tgt = lax.rem(my + d, N_DEV)
            rdma = pltpu.make_async_remote_copy(
                src_ref=x_ref.at[pl.ds(tgt * m_per, m_per), :],
                dst_ref=comm_ref.at[my],
                send_sem=send_sems.at[d - 1],
                recv_sem=recv_sems.at[my],
                device_id=(tgt,),
                device_id_type=pl.DeviceIdType.MESH,
            )
            rdma.start()
            rdmas.append(rdma)

        for g in range(4):
            for s in range(4 * g, 4 * g + 4):
                @pl.when(s != my)
                def _():
                    recv = pltpu.make_async_remote_copy(
                        src_ref=x_ref.at[pl.ds(0, m_per), :],
                        dst_ref=comm_ref.at[s],
                        send_sem=send_sems.at[0],
                        recv_sem=recv_sems.at[s],
                        device_id=(my,),
                        device_id_type=pl.DeviceIdType.MESH,
                    )
                    recv.wait_recv()
            w_loads[g].wait()
            xg = jnp.concatenate(
                [comm_ref[s] for s in range(4 * g, 4 * g + 4)], axis=1
            )
            partial = jnp.dot(
                xg,
                w_ref[pl.ds(g * k_chunk, k_chunk), :],
                preferred_element_type=jnp.float32,
            )
            if g == 0:
                out_ref[:, :] = partial
            else:
                out_ref[:, :] += partial

        out_store = pltpu.make_async_copy(out_ref, out_hbm_ref, out_sem)
        out_store.start()

        for d in range(1, N_DEV):
            rdmas[d - 1].wait_send()
        out_store.wait()

    return pl.pallas_call(
        body,
        out_shape=jax.ShapeDtypeStruct((m_per, n), jnp.float32),
        in_specs=[
            pl.BlockSpec(memory_space=pl.ANY),
            pl.BlockSpec(memory_space=pl.ANY),
        ],
        out_specs=pl.BlockSpec(memory_space=pl.ANY),
        scratch_shapes=[
            pltpu.VMEM((k_total, k_per), x.dtype),
            pltpu.VMEM((k_total, n), w_mat.dtype),
            pltpu.VMEM((m_per, n), jnp.float32),
            pltpu.VMEM((N_DEV, m_per, k_per), x.dtype),
            pltpu.SemaphoreType.DMA((N_DEV - 1,)),
            pltpu.SemaphoreType.DMA((N_DEV,)),
            pltpu.SemaphoreType.DMA((4,)),
            pltpu.SemaphoreType.DMA,
            pltpu.SemaphoreType.DMA,
        ],
        compiler_params=pltpu.CompilerParams(collective_id=0),
    )(x, w_mat)
